# baseline (speedup 1.0000x reference)
import jax
import jax.numpy as jnp
from jax import lax
from jax.experimental import pallas as pl
from jax.experimental.pallas import tpu as pltpu
from jax.experimental.pallas import tpu_sc as plsc

VOCAB = 30522
SEQ = 128
DIM = 256
LANES = 16
NC = 2
NS = 16
NW = NC * NS
SB = 4
PB = 8
WIN = SB * PB
NBUF = 8
LAG = 3


def kernel(x, tok_table, pos_table):
    batch, seq = x.shape
    n = batch * seq
    seq_per_w = batch // NW
    npb = seq // PB
    nwin = (seq_per_w // SB) * npb

    x4 = (
        x.astype(jnp.int32)
        .reshape(NW, seq_per_w // SB, SB, npb, PB)
        .transpose(0, 1, 3, 2, 4)
        .reshape(NW, nwin, WIN)
    )

    mesh = plsc.VectorSubcoreMesh(core_axis_name="core", subcore_axis_name="subcore")

    @pl.kernel(
        out_type=jax.ShapeDtypeStruct((n, DIM), jnp.float32),
        mesh=mesh,
        scratch_types=(
            [
                pltpu.VMEM((SEQ, DIM), jnp.float32),
                pltpu.VMEM((nwin, WIN), jnp.int32),
            ]
            + [pltpu.VMEM((WIN, DIM), jnp.float32) for _ in range(NBUF)]
            + [pltpu.SemaphoreType.DMA for _ in range(2 * NBUF)]
        ),
    )
    def emb_kernel(tok_hbm, i_hbm, pos_hbm, o_hbm, pos_v, idx_v, *rest):
        bufs = rest[:NBUF]
        gsems = rest[NBUF:2 * NBUF]
        ssems = rest[2 * NBUF:]

        wid = lax.axis_index("subcore") * NC + lax.axis_index("core")

        pltpu.sync_copy(i_hbm.at[wid], idx_v)
        pltpu.sync_copy(pos_hbm, pos_v)

        def gather(w, b):
            pltpu.async_copy(tok_hbm.at[idx_v.at[w]], bufs[b], gsems[b])

        def gather_wait(w, b):
            pltpu.make_async_copy(
                tok_hbm.at[idx_v.at[w]], bufs[b], gsems[b]
            ).wait()

        def _store_slices(w, b, i):
            si = w // npb
            p0 = (w % npb) * PB
            row0 = (wid * seq_per_w + si * SB + i) * seq + p0
            return bufs[b].at[pl.ds(i * PB, PB), :], o_hbm.at[pl.ds(row0, PB), :]

        def store(w, b):
            for i in range(SB):
                src, dst = _store_slices(w, b, i)
                pltpu.async_copy(src, dst, ssems[b])

        def store_wait(w, b):
            pltpu.make_async_copy(
                tok_hbm.at[pl.ds(0, WIN), :], bufs[b], ssems[b]
            ).wait()

        def add_pos(w, b):
            g = bufs[b]
            p0 = (w % npb) * PB

            @pl.loop(0, PB)
            def _(q):
                pv = [
                    pos_v.at[p0 + q, pl.ds(j * LANES, LANES)][...]
                    for j in range(DIM // LANES)
                ]

                @plsc.parallel_loop(0, SB, unroll=SB)
                def _(i):
                    r = i * PB + q
                    for j in range(DIM // LANES):
                        slc = pl.ds(j * LANES, LANES)
                        g.at[r, slc][...] = g.at[r, slc][...] + pv[j]

        def process(w, b):
            gather_wait(w, b)
            add_pos(w, b)
            store(w, b)

        for b in range(NBUF):
            gather(b, b)
        for w in range(LAG):
            process(w, w % NBUF)

        @pl.loop(0, nwin - NBUF, step=NBUF)
        def _(w0):
            for b in range(NBUF):
                process(w0 + b + LAG, (b + LAG) % NBUF)
                store_wait(w0 + b, b)
                gather(w0 + b + NBUF, b)

        for k in range(NBUF - LAG):
            w = nwin - NBUF + LAG + k
            process(w, w % NBUF)
        for k in range(NBUF):
            w = nwin - NBUF + k
            store_wait(w, w % NBUF)

    out = emb_kernel(tok_table, x4, pos_table)
    return out.reshape(batch, seq, DIM)
